# baseline (speedup 1.0000x reference)
import jax
import jax.numpy as jnp
from jax import lax
from jax.experimental import pallas as pl
from jax.experimental.pallas import tpu as pltpu
from jax.experimental.pallas import tpu_sc as plsc

B, S, D = 4096, 200, 64
N = B * S
LW = 128
NW = 32
PER_W = N // NW
IROWS_W = PER_W // LW
CHUNK = 256
KSUB = CHUNK // LW
NCH = PER_W // CHUNK
NBUF = 4
K = 2
NTOT = 2 * NCH
NT = (NTOT + NBUF - 1) // NBUF
IDX_ROWS = N // LW


def _body(idx1, idx2, wq, ws, out1, out2, idx_v1, idx_v2, r0, r1, r2, r3,
          sg0, sg1, sg2, sg3, sw0, sw1, sw2, sw3, si):
    cid = lax.axis_index("c")
    sid = lax.axis_index("s")
    wid = sid * 2 + cid
    rows = (r0, r1, r2, r3)
    semg = (sg0, sg1, sg2, sg3)
    semw = (sw0, sw1, sw2, sw3)
    base = wid * PER_W

    pltpu.async_copy(idx2.at[pl.ds(wid * IROWS_W, IROWS_W)], idx_v2, si)
    pltpu.sync_copy(idx1.at[pl.ds(wid * IROWS_W, IROWS_W)], idx_v1)

    def fire_g(idx_v, tbl, i, b):
        for j in range(KSUB):
            pltpu.async_copy(
                tbl.at[idx_v.at[i * KSUB + j]],
                rows[b].at[pl.ds(j * LW, LW)],
                semg[b],
            )

    def drain_g(b):
        pltpu.make_async_copy(wq.at[pl.ds(0, CHUNK)], rows[b], semg[b]).wait()

    def wb_fire(out_hbm, i, b):
        pltpu.async_copy(rows[b], out_hbm.at[pl.ds(base + i * CHUNK, CHUNK)],
                         semw[b])

    def wb_wait(b):
        pltpu.make_async_copy(
            rows[b], out1.at[pl.ds(0, CHUNK)], semw[b]).wait()

    for b in range(K):
        fire_g(idx_v1, wq, b, b)

    def step(t, carry):
        for j in range(NBUF):
            g = NBUF * t + j
            b = j
            bf = (j + K) % NBUF
            gf = g + K

            @pl.when(g < NTOT)
            def _():
                drain_g(b)

            @pl.when(g < NCH)
            def _():
                wb_fire(out1, g, b)

            @pl.when(jnp.logical_and(g >= NCH, g < NTOT))
            def _():
                wb_fire(out2, g - NCH, b)

            @pl.when(jnp.logical_and(gf < NTOT, g >= NBUF - K))
            def _():
                wb_wait(bf)

            @pl.when(gf == NCH)
            def _():
                pltpu.make_async_copy(
                    idx2.at[pl.ds(0, IROWS_W)], idx_v2, si).wait()

            @pl.when(gf < NCH)
            def _():
                fire_g(idx_v1, wq, gf, bf)

            @pl.when(jnp.logical_and(gf >= NCH, gf < NTOT))
            def _():
                fire_g(idx_v2, ws, gf - NCH, bf)

        return carry

    lax.fori_loop(0, NT, step, 0)
    for b in range(NBUF):
        wb_wait(b)


@jax.jit
def _emb(i1, i2, wq, ws):
    mesh = plsc.VectorSubcoreMesh(core_axis_name="c", subcore_axis_name="s")
    f = pl.kernel(
        _body,
        mesh=mesh,
        out_type=[
            jax.ShapeDtypeStruct((N, D), jnp.float32),
            jax.ShapeDtypeStruct((N, D), jnp.float32),
        ],
        scratch_types=[
            pltpu.VMEM((IROWS_W, LW), jnp.int32),
            pltpu.VMEM((IROWS_W, LW), jnp.int32),
        ] + [pltpu.VMEM((CHUNK, D), jnp.float32)] * NBUF
          + [pltpu.SemaphoreType.DMA] * (2 * NBUF + 1),
        compiler_params=pltpu.CompilerParams(use_tc_tiling_on_sc=False),
    )
    return f(i1, i2, wq, ws)


def kernel(input_1, input_2, Wq, Ws):
    wq = Wq.at[0].set(0.0)
    ws = Ws.at[0].set(0.0)
    i1 = input_1.reshape(IDX_ROWS, LW).astype(jnp.int32)
    i2 = input_2.reshape(IDX_ROWS, LW).astype(jnp.int32)
    o1, o2 = _emb(i1, i2, wq, ws)
    return o1.reshape(B, S, D), o2.reshape(B, S, D)
